# baseline (speedup 1.0000x reference)
import functools

import jax
import jax.numpy as jnp
from jax import lax
from jax.experimental import pallas as pl
from jax.experimental.pallas import tpu as pltpu
from jax.experimental.pallas import tpu_sc as plsc

EMBED_DIM = 300
PAD_DIM = 304
NUM_CORES = 2
NUM_SUBCORES = 16
NUM_WORKERS = NUM_CORES * NUM_SUBCORES
NI, NJ = 4096, 200
IT = NI // 128
JT = NJ // 8
LH = 64


def _make_gather():
  mesh = plsc.VectorSubcoreMesh(core_axis_name="c", subcore_axis_name="s")

  @functools.partial(
      pl.kernel,
      mesh=mesh,
      out_type=jax.ShapeDtypeStruct((EMBED_DIM, JT, IT, 8, 128), jnp.float32),
      scratch_types=[
          pltpu.VMEM((LH,), jnp.int32),
          pltpu.VMEM((LH,), jnp.int32),
          pltpu.VMEM((LH,), jnp.int32),
          pltpu.VMEM((LH,), jnp.int32),
          pltpu.VMEM((LH, PAD_DIM), jnp.float32),
          pltpu.VMEM((LH, PAD_DIM), jnp.float32),
          pltpu.VMEM((EMBED_DIM, LH), jnp.float32),
          pltpu.VMEM((EMBED_DIM, LH), jnp.float32),
          pltpu.SemaphoreType.DMA,
          pltpu.SemaphoreType.DMA,
          pltpu.SemaphoreType.DMA,
          pltpu.SemaphoreType.DMA,
          pltpu.SemaphoreType.DMA,
          pltpu.SemaphoreType.DMA,
      ],
      compiler_params=pltpu.CompilerParams(
          use_tc_tiling_on_sc=False, needs_layout_passes=False),
  )
  def gather_kernel(xt_hbm, table_hbm, out_hbm,
                    idx00, idx01, idx10, idx11,
                    rows0, rows1, stg0, stg1,
                    isem0, isem1, gsem0, gsem1, wsem0, wsem1):
    wid = lax.axis_index("s") * NUM_CORES + lax.axis_index("c")
    it = wid
    idxs = ((idx00, idx01), (idx10, idx11))
    rows = (rows0, rows1)
    stgs = (stg0, stg1)
    isems = (isem0, isem1)
    gsems = (gsem0, gsem1)
    wsems = (wsem0, wsem1)
    iota16 = lax.iota(jnp.int32, 16)

    def x_slice(jcol, u):
      return xt_hbm.at[jcol, pl.ds(it * 128 + u * LH, LH)]

    def out_slice(jcol, u):
      jt = jcol // 8
      s = jcol % 8
      return out_hbm.at[pl.ds(0, EMBED_DIM), jt, it, s, pl.ds(u * LH, LH)]

    def transpose_rows(u):
      rbuf = rows[u]
      sbuf = stgs[u]

      def body(e, carry):
        col = jnp.full((16,), e, jnp.int32)
        for g in range(LH // 16):
          vec = plsc.load_gather(rbuf, [g * 16 + iota16, col])
          sbuf[e, pl.ds(g * 16, 16)] = vec
        return carry

      lax.fori_loop(0, EMBED_DIM, body, 0)

    for u in (0, 1):
      pltpu.sync_copy(x_slice(0, u), idxs[0][u])
      pltpu.async_copy(table_hbm.at[idxs[0][u]], rows[u], gsems[u])
      pltpu.async_copy(x_slice(1, u), idxs[1][u], isems[u])

    def half_loop(jcol, parity, u):
      @pl.when(jcol >= 1)
      def _():
        pltpu.make_async_copy(
            stgs[u], out_slice(jcol - 1, u), wsems[u]).wait()
      pltpu.make_async_copy(
          table_hbm.at[idxs[parity][u]], rows[u], gsems[u]).wait()
      transpose_rows(u)
      pltpu.async_copy(stgs[u], out_slice(jcol, u), wsems[u])

      @pl.when(jcol < NJ - 1)
      def _():
        pltpu.make_async_copy(
            x_slice(jcol + 1, u), idxs[1 - parity][u], isems[u]).wait()
        pltpu.async_copy(table_hbm.at[idxs[1 - parity][u]], rows[u],
                         gsems[u])

        @pl.when(jcol < NJ - 2)
        def _():
          pltpu.async_copy(x_slice(jcol + 2, u), idxs[parity][u], isems[u])

    def loop_body(jc2, carry):
      for par in (0, 1):
        jcol = jc2 * 2 + par
        for u in (0, 1):
          half_loop(jcol, par, u)
      return carry

    lax.fori_loop(0, NJ // 2, loop_body, 0)

    for u in (0, 1):
      pltpu.make_async_copy(
          stgs[u], out_slice(NJ - 1, u), wsems[u]).wait()

  return gather_kernel


def kernel(x, table):
  table_p = jnp.pad(table, ((0, 0), (0, PAD_DIM - EMBED_DIM)))
  out5d = _make_gather()(x.T, table_p)
  return out5d.transpose(2, 4, 1, 3, 0).reshape(NI, NJ, EMBED_DIM)
